# Initial kernel scaffold; baseline (speedup 1.0000x reference)
import jax
import jax.numpy as jnp
from jax.experimental import pallas as pl


def kernel(hidden_states, gate_w, expert_gate_up, expert_down, shared_gate_up, shared_down):
    raise NotImplementedError("write your pallas kernel here")


# speedup vs baseline: 1.2684x; 1.2684x over previous
import functools

import jax
import jax.numpy as jnp
from jax.experimental import pallas as pl
from jax.experimental.pallas import tpu as pltpu

T = 2048
D = 1024
E = 8
K = 2
F = 512
SF = 512

BT = 1024


def _moe_kernel(x_ref, gate_w_ref, gu_ref, down_ref, sgu_ref, sdown_ref,
                out_ref, combine_ref):
    e = pl.program_id(1)
    x = x_ref[...]

    @pl.when(e == 0)
    def _router_and_shared():
        logits = jnp.dot(x, gate_w_ref[...].T, preferred_element_type=jnp.float32)
        logits = logits - jnp.max(logits, axis=-1, keepdims=True)
        ex = jnp.exp(logits)
        probs = ex / jnp.sum(ex, axis=-1, keepdims=True)
        a1 = jnp.argmax(probs, axis=-1)
        m1 = jnp.max(probs, axis=-1)
        col = jax.lax.broadcasted_iota(jnp.int32, (BT, E), 1)
        masked = jnp.where(col == a1[:, None], -jnp.inf, probs)
        a2 = jnp.argmax(masked, axis=-1)
        m2 = jnp.max(masked, axis=-1)
        s = m1 + m2
        w1 = (m1 / s)[:, None]
        w2 = (m2 / s)[:, None]
        combine_ref[...] = jnp.where(col == a1[:, None], w1, 0.0) + jnp.where(
            col == a2[:, None], w2, 0.0)

        sh = jnp.dot(x, sgu_ref[...], preferred_element_type=jnp.float32)
        sg = sh[:, :SF]
        su = sh[:, SF:]
        act = (sg * jax.nn.sigmoid(sg)) * su
        out_ref[...] = jnp.dot(act, sdown_ref[...], preferred_element_type=jnp.float32)

    h = jnp.dot(x, gu_ref[0], preferred_element_type=jnp.float32)
    g = h[:, :F]
    u = h[:, F:]
    act = (g * jax.nn.sigmoid(g)) * u
    eo = jnp.dot(act, down_ref[0], preferred_element_type=jnp.float32)
    col = jax.lax.broadcasted_iota(jnp.int32, (BT, E), 1)
    w = jnp.sum(jnp.where(col == e, combine_ref[...], 0.0), axis=-1,
                keepdims=True)
    out_ref[...] += w * eo


@jax.jit
def kernel(hidden_states, gate_w, expert_gate_up, expert_down, shared_gate_up,
           shared_down):
    grid = (T // BT, E)
    return pl.pallas_call(
        _moe_kernel,
        grid=grid,
        in_specs=[
            pl.BlockSpec((BT, D), lambda t, e: (t, 0)),
            pl.BlockSpec((E, D), lambda t, e: (0, 0)),
            pl.BlockSpec((1, D, 2 * F), lambda t, e: (e, 0, 0)),
            pl.BlockSpec((1, F, D), lambda t, e: (e, 0, 0)),
            pl.BlockSpec((D, 2 * SF), lambda t, e: (0, 0)),
            pl.BlockSpec((SF, D), lambda t, e: (0, 0)),
        ],
        out_specs=pl.BlockSpec((BT, D), lambda t, e: (t, 0)),
        out_shape=jax.ShapeDtypeStruct((T, D), jnp.float32),
        scratch_shapes=[pltpu.VMEM((BT, E), jnp.float32)],
        compiler_params=pltpu.CompilerParams(
            dimension_semantics=("arbitrary", "arbitrary"),
        ),
    )(hidden_states, gate_w, expert_gate_up, expert_down, shared_gate_up,
      shared_down)
